# baseline (speedup 1.0000x reference)
import jax
import jax.numpy as jnp
from jax.experimental import pallas as pl
from jax.experimental.pallas import tpu as pltpu

K = 16
KK = 32


def _knn_body(x_ref, out_ref, s0_ref, s1_ref):
    b = pl.program_id(0)
    odd = (b % 2) == 1

    xb = x_ref[0]
    C, N = xb.shape
    norm = jnp.sqrt(jnp.sum(xb * xb, axis=0, keepdims=True))
    xn = xb / jnp.maximum(norm, 1e-12)
    inner = jax.lax.dot_general(
        xn, xn,
        dimension_numbers=(((0,), (0,)), ((), ())),
        preferred_element_type=jnp.float32,
    )
    x_inner = -2.0 * inner
    sq = jnp.sum(xn * xn, axis=0, keepdims=True)
    dist = (jnp.transpose(sq) + x_inner) + sq
    snew = -dist

    sA = s0_ref[...]
    sB = s1_ref[...]
    score = jnp.where(odd, sA, sB)
    col = jax.lax.broadcasted_iota(jnp.int32, (N, N), 1)
    neg_inf = jnp.float32(-jnp.inf)
    cols_out = []
    for k in range(KK):
        idx = jnp.argmax(score, axis=1, keepdims=True).astype(jnp.int32)
        if k % 2 == 0:
            cols_out.append(idx)
        if k != KK - 1:
            score = jnp.where(col == idx, neg_inf, score)
    out_ref[0] = jnp.concatenate(cols_out, axis=1)

    s0_ref[...] = jnp.where(odd, sA, snew)
    s1_ref[...] = jnp.where(odd, snew, sB)


@jax.jit
def kernel(x):
    B, C, N, _ = x.shape
    xs = jnp.squeeze(x, -1)
    nn_idx = pl.pallas_call(
        _knn_body,
        grid=(B + 1,),
        in_specs=[
            pl.BlockSpec((1, C, N), lambda b: (jnp.minimum(b, B - 1), 0, 0))
        ],
        out_specs=pl.BlockSpec(
            (1, N, K), lambda b: (jnp.maximum(b - 1, 0), 0, 0)
        ),
        out_shape=jax.ShapeDtypeStruct((B, N, K), jnp.int32),
        scratch_shapes=[
            pltpu.VMEM((N, N), jnp.float32),
            pltpu.VMEM((N, N), jnp.float32),
        ],
    )(xs)
    center_idx = jnp.broadcast_to(
        jnp.arange(N, dtype=jnp.int32)[None, :, None], (B, N, K)
    )
    return jnp.stack((nn_idx, center_idx), axis=0)


# speedup vs baseline: 1.1352x; 1.1352x over previous
import jax
import jax.numpy as jnp
from jax.experimental import pallas as pl

K = 16
KK = 32


def _knn_body(x_ref, out_ref):
    xb = x_ref[0]
    C, N = xb.shape
    norm = jnp.sqrt(jnp.sum(xb * xb, axis=0, keepdims=True))
    xn = xb / jnp.maximum(norm, 1e-12)
    inner = jax.lax.dot_general(
        xn, xn,
        dimension_numbers=(((0,), (0,)), ((), ())),
        preferred_element_type=jnp.float32,
    )
    x_inner = -2.0 * inner
    sq = jnp.sum(xn * xn, axis=0, keepdims=True)
    dist = (jnp.transpose(sq) + x_inner) + sq
    score = -dist
    col = jax.lax.broadcasted_iota(jnp.int32, (N, N), 1)
    neg_inf = jnp.float32(-jnp.inf)
    cols_out = []
    for k in range(KK):
        idx = jnp.argmax(score, axis=1, keepdims=True).astype(jnp.int32)
        if k % 2 == 0:
            cols_out.append(idx)
        if k != KK - 1:
            score = jnp.where(col == idx, neg_inf, score)
    out_ref[0] = jnp.concatenate(cols_out, axis=1)


@jax.jit
def kernel(x):
    B, C, N, _ = x.shape
    xs = jnp.squeeze(x, -1)
    nn_idx = pl.pallas_call(
        _knn_body,
        grid=(B,),
        in_specs=[pl.BlockSpec((1, C, N), lambda b: (b, 0, 0))],
        out_specs=pl.BlockSpec((1, N, K), lambda b: (b, 0, 0)),
        out_shape=jax.ShapeDtypeStruct((B, N, K), jnp.int32),
    )(xs)
    center_idx = jnp.broadcast_to(
        jnp.arange(N, dtype=jnp.int32)[None, :, None], (B, N, K)
    )
    return jnp.stack((nn_idx, center_idx), axis=0)
